# baseline (speedup 1.0000x reference)
import jax
import jax.numpy as jnp
from jax import lax
from jax.experimental import pallas as pl
from jax.experimental.pallas import tpu as pltpu
from jax.experimental.pallas import tpu_sc as plsc

BATCH = 4096
SEQLEN = 200
EMBED = 64

L0 = 160
BL = 8

NC = 2
NS = 16
NW = NC * NS
DT = EMBED // 8
UNIT = 8 * BATCH
SC_UNITS = (SEQLEN - L0) * DT
UNITS_PER_W = SC_UNITS // NW
U_BASE = L0 * DT
XROWS = 8


def _sc_body(xt_hbm, pos_hbm, out_hbm, xt_v, pos_v, buf0, buf1, sem0, sem1):
    wid = lax.axis_index("s") * NC + lax.axis_index("c")
    u_base = U_BASE + wid * UNITS_PER_W
    l_base = lax.min(u_base // DT, SEQLEN - XROWS)

    pltpu.sync_copy(xt_hbm.at[pl.ds(l_base, XROWS)], xt_v)
    pltpu.sync_copy(pos_hbm.at[pl.ds(l_base, XROWS)], pos_v)

    def compute_unit(buf, u):
        l_loc = u // DT - l_base
        dt = u % DT
        g = pl.multiple_of((dt // 2) * 16, 16)
        pv = pos_v[l_loc, pl.ds(g, 16)]
        lo = (dt % 2) == 0
        splats = [jnp.where(lo, pv[dd], pv[8 + dd]) for dd in range(8)]

        def bt_body(bth, carry):
            for h in range(2):
                bt = bth * 2 + h
                boff = pl.multiple_of(bt * 128, 128)
                for c in range(8):
                    xv = xt_v[l_loc, pl.ds(boff + c * 16, 16)]
                    for dd in range(8):
                        buf[pl.ds(boff * 8 + dd * 128 + c * 16, 16)] = (
                            xv + splats[dd]
                        )
            return carry

        lax.fori_loop(0, 16, bt_body, 0)

    def body(i, carry):
        u0 = u_base + 2 * i

        @pl.when(i > 0)
        def _():
            pltpu.make_async_copy(buf0, out_hbm.at[pl.ds(0, UNIT)], sem0).wait()

        compute_unit(buf0, u0)
        pltpu.make_async_copy(buf0, out_hbm.at[pl.ds(u0 * UNIT, UNIT)], sem0).start()

        @pl.when(i > 0)
        def _():
            pltpu.make_async_copy(buf1, out_hbm.at[pl.ds(0, UNIT)], sem1).wait()

        compute_unit(buf1, u0 + 1)
        pltpu.make_async_copy(
            buf1, out_hbm.at[pl.ds((u0 + 1) * UNIT, UNIT)], sem1
        ).start()
        return carry

    lax.fori_loop(0, UNITS_PER_W // 2, body, 0)
    pltpu.make_async_copy(buf0, out_hbm.at[pl.ds(0, UNIT)], sem0).wait()
    pltpu.make_async_copy(buf1, out_hbm.at[pl.ds(0, UNIT)], sem1).wait()


def _tc_body(sc_ref, x_ref, pos_ref, out_ref):
    for i in range(BL):
        x2 = x_ref[i]
        p = pos_ref[pl.program_id(0) * BL + i]
        pt = jnp.broadcast_to(p[None, :], (128, 64)).T
        xe = x2[None, :, None, :]
        pe = pt.reshape(8, 8, 128)[:, None, :, :]
        out_ref[i] = xe + pe


def kernel(x, pos_table):
    xt = x.T
    mesh = plsc.VectorSubcoreMesh(core_axis_name="c", subcore_axis_name="s")
    sc_k = pl.kernel(
        _sc_body,
        mesh=mesh,
        compiler_params=pltpu.CompilerParams(use_tc_tiling_on_sc=False),
        out_type=jax.ShapeDtypeStruct((SEQLEN * DT * UNIT,), jnp.float32),
        scratch_types=[
            pltpu.VMEM((XROWS, BATCH), jnp.float32),
            pltpu.VMEM((XROWS, EMBED), jnp.float32),
            pltpu.VMEM((UNIT,), jnp.float32),
            pltpu.VMEM((UNIT,), jnp.float32),
            pltpu.SemaphoreType.DMA,
            pltpu.SemaphoreType.DMA,
        ],
    )
    sc_flat = sc_k(xt, pos_table)
    sc5 = sc_flat.reshape(SEQLEN, DT, BATCH // 128, 8, 128)

    xr = xt.reshape(SEQLEN, 32, 128)
    out5 = pl.pallas_call(
        _tc_body,
        grid=(L0 // BL,),
        in_specs=[
            pl.BlockSpec(memory_space=pl.ANY),
            pl.BlockSpec((BL, 32, 128), lambda l: (l, 0, 0)),
            pl.BlockSpec((SEQLEN, EMBED), lambda l: (0, 0)),
        ],
        out_specs=pl.BlockSpec((BL, 8, 32, 8, 128), lambda l: (l, 0, 0, 0, 0)),
        out_shape=jax.ShapeDtypeStruct((SEQLEN, DT, BATCH // 128, 8, 128), jnp.float32),
        input_output_aliases={0: 0},
    )(sc5, xr, pos_table)
    return out5.transpose(2, 4, 0, 1, 3).reshape(BATCH, SEQLEN, EMBED)


# speedup vs baseline: 1.0337x; 1.0337x over previous
import jax
import jax.numpy as jnp
from jax import lax
from jax.experimental import pallas as pl
from jax.experimental.pallas import tpu as pltpu
from jax.experimental.pallas import tpu_sc as plsc

BATCH = 4096
SEQLEN = 200
EMBED = 64

L0 = 184
BL = 8

NC = 2
NS = 16
NW = NC * NS
DT = EMBED // 8
UNIT = 8 * BATCH
SC_UNITS = (SEQLEN - L0) * DT
UNITS_PER_W = SC_UNITS // NW
U_BASE = L0 * DT
XROWS = 8


def _sc_body(xt_hbm, pos_hbm, out_hbm, xt_v, pos_v, buf0, buf1, sem0, sem1):
    wid = lax.axis_index("s") * NC + lax.axis_index("c")
    u_base = U_BASE + wid * UNITS_PER_W
    l_base = lax.min(u_base // DT, SEQLEN - XROWS)

    pltpu.sync_copy(xt_hbm.at[pl.ds(l_base, XROWS)], xt_v)
    pltpu.sync_copy(pos_hbm.at[pl.ds(l_base, XROWS)], pos_v)

    def compute_unit(buf, u):
        l_loc = u // DT - l_base
        dt = u % DT
        g = pl.multiple_of((dt // 2) * 16, 16)
        pv = pos_v[l_loc, pl.ds(g, 16)]
        lo = (dt % 2) == 0
        splats = [jnp.where(lo, pv[dd], pv[8 + dd]) for dd in range(8)]

        def bt_body(bth, carry):
            for h in range(2):
                bt = bth * 2 + h
                boff = pl.multiple_of(bt * 128, 128)
                for c in range(8):
                    xv = xt_v[l_loc, pl.ds(boff + c * 16, 16)]
                    for dd in range(8):
                        buf[pl.ds(boff * 8 + dd * 128 + c * 16, 16)] = (
                            xv + splats[dd]
                        )
            return carry

        lax.fori_loop(0, 16, bt_body, 0)

    def body(i, carry):
        u0 = u_base + 2 * i

        @pl.when(i > 0)
        def _():
            pltpu.make_async_copy(buf0, out_hbm.at[pl.ds(0, UNIT)], sem0).wait()

        compute_unit(buf0, u0)
        pltpu.make_async_copy(buf0, out_hbm.at[pl.ds(u0 * UNIT, UNIT)], sem0).start()

        @pl.when(i > 0)
        def _():
            pltpu.make_async_copy(buf1, out_hbm.at[pl.ds(0, UNIT)], sem1).wait()

        compute_unit(buf1, u0 + 1)
        pltpu.make_async_copy(
            buf1, out_hbm.at[pl.ds((u0 + 1) * UNIT, UNIT)], sem1
        ).start()
        return carry

    lax.fori_loop(0, UNITS_PER_W // 2, body, 0)
    pltpu.make_async_copy(buf0, out_hbm.at[pl.ds(0, UNIT)], sem0).wait()
    pltpu.make_async_copy(buf1, out_hbm.at[pl.ds(0, UNIT)], sem1).wait()


def _tc_body(sc_ref, x_ref, pos_ref, out_ref):
    for i in range(BL):
        x2 = x_ref[i]
        p = pos_ref[pl.program_id(0) * BL + i]
        pt = jnp.broadcast_to(p[None, :], (128, 64)).T
        xe = x2[None, :, None, :]
        pe = pt.reshape(8, 8, 128)[:, None, :, :]
        out_ref[i] = xe + pe


def kernel(x, pos_table):
    xt = x.T
    mesh = plsc.VectorSubcoreMesh(core_axis_name="c", subcore_axis_name="s")
    sc_k = pl.kernel(
        _sc_body,
        mesh=mesh,
        compiler_params=pltpu.CompilerParams(use_tc_tiling_on_sc=False),
        out_type=jax.ShapeDtypeStruct((SEQLEN * DT * UNIT,), jnp.float32),
        scratch_types=[
            pltpu.VMEM((XROWS, BATCH), jnp.float32),
            pltpu.VMEM((XROWS, EMBED), jnp.float32),
            pltpu.VMEM((UNIT,), jnp.float32),
            pltpu.VMEM((UNIT,), jnp.float32),
            pltpu.SemaphoreType.DMA,
            pltpu.SemaphoreType.DMA,
        ],
    )
    sc_flat = sc_k(xt, pos_table)
    sc5 = sc_flat.reshape(SEQLEN, DT, BATCH // 128, 8, 128)

    xr = xt.reshape(SEQLEN, 32, 128)
    out5 = pl.pallas_call(
        _tc_body,
        grid=(L0 // BL,),
        in_specs=[
            pl.BlockSpec(memory_space=pl.ANY),
            pl.BlockSpec((BL, 32, 128), lambda l: (l, 0, 0)),
            pl.BlockSpec((SEQLEN, EMBED), lambda l: (0, 0)),
        ],
        out_specs=pl.BlockSpec((BL, 8, 32, 8, 128), lambda l: (l, 0, 0, 0, 0)),
        out_shape=jax.ShapeDtypeStruct((SEQLEN, DT, BATCH // 128, 8, 128), jnp.float32),
        input_output_aliases={0: 0},
    )(sc5, xr, pos_table)
    return out5.transpose(2, 4, 0, 1, 3).reshape(BATCH, SEQLEN, EMBED)
